# baseline (speedup 1.0000x reference)
import functools

import jax
import jax.numpy as jnp
from jax import lax
from jax.experimental import pallas as pl
from jax.experimental.pallas import tpu as pltpu
from jax.experimental.pallas import tpu_sc as plsc

N = 10000
E = 320000
NC = 2
NS = 16
NW = NC * NS
B = 80
NB = 125
D = 5
NPAD = 10240
RPT = NPAD // NS
F1 = 48
F2 = 32

_vector_mesh = plsc.VectorSubcoreMesh(core_axis_name="c", subcore_axis_name="s")


def _make_agg(F):
  def body(y_hbm, src_hbm, dst_hbm, out_hbm, sidx, didx, b0, b1, b2, b3, b4,
           zbuf, ysp, acc, g0, g1, g2, g3, g4, s0, s1, s2, s3, s4):
    c = lax.axis_index("c")
    s = lax.axis_index("s")
    wid = c * NS + s
    bufs = [b0, b1, b2, b3, b4]
    gsems = [g0, g1, g2, g3, g4]
    ssems = [s0, s1, s2, s3, s4]
    zero16 = jnp.zeros((16,), jnp.float32)

    @pl.loop(0, 128)
    def _zero(i):
      for j in range(F // 16):
        zbuf[i, pl.ds(j * 16, 16)] = zero16

    for k in range(RPT // 128):
      pltpu.sync_copy(zbuf, acc.at[pl.ds(s * RPT + k * 128, 128)])

    ycp = pltpu.async_copy(y_hbm.at[pl.ds(s * (N // NS), N // NS)],
                           ysp.at[pl.ds(s * (N // NS), N // NS)], g0)
    pltpu.sync_copy(src_hbm.at[wid], sidx)
    pltpu.sync_copy(dst_hbm.at[wid], didx)
    ycp.wait()
    plsc.subcore_barrier()

    def gather(j, t):
      return pltpu.make_async_copy(ysp.at[sidx.at[j]], bufs[t], gsems[t])

    def scatter(j, t):
      return pltpu.async_copy(bufs[t], acc.at[didx.at[j]], ssems[t], add=True)

    def drain_scatter(t):
      pltpu.make_async_copy(bufs[t], acc.at[didx.at[0]], ssems[t]).wait()

    pltpu.async_copy(ysp.at[sidx.at[0]], bufs[0], gsems[0])

    @pl.loop(0, NB // D)
    def _grp(k):
      j0 = k * D
      for t in range(D):
        tn = (t + 1) % D
        j = j0 + t
        jn = j + 1

        @pl.when(jn < NB)
        def _pref():
          @pl.when(jn >= D)
          def _drain():
            drain_scatter(tn)
          pltpu.async_copy(ysp.at[sidx.at[jn]], bufs[tn], gsems[tn])

        gather(j, t).wait()
        scatter(j, t)

    for t in range(D):
      drain_scatter(t)

    plsc.subcore_barrier()
    for k in range(RPT // 128):
      sl = pl.ds(s * RPT + k * 128, 128)
      pltpu.sync_copy(acc.at[sl], out_hbm.at[c, sl])

  return pl.kernel(
      body,
      out_type=jax.ShapeDtypeStruct((NC, NPAD, F), jnp.float32),
      mesh=_vector_mesh,
      compiler_params=pltpu.CompilerParams(use_tc_tiling_on_sc=False),
      scratch_types=(
          [pltpu.VMEM((NB, B), jnp.int32),
           pltpu.VMEM((NB, B), jnp.int32)] +
          [pltpu.VMEM((B, F), jnp.float32)] * D +
          [pltpu.VMEM((128, F), jnp.float32),
           pltpu.VMEM_SHARED((N, F), jnp.float32),
           pltpu.VMEM_SHARED((NPAD, F), jnp.float32)] +
          [pltpu.SemaphoreType.DMA] * (2 * D)
      ),
  )


def _make_deg():
  def body(dst_hbm, out_hbm, didx, ones, zbuf, acc, ssem):
    c = lax.axis_index("c")
    s = lax.axis_index("s")
    wid = c * NS + s
    zero16 = jnp.zeros((16,), jnp.float32)
    one16 = jnp.ones((16,), jnp.float32)

    @pl.loop(0, 128)
    def _fill(i):
      zbuf[i] = zero16

    @pl.loop(0, B)
    def _fill1(i):
      ones[i] = one16

    for k in range(RPT // 128):
      pltpu.sync_copy(zbuf, acc.at[pl.ds(s * RPT + k * 128, 128)])

    pltpu.sync_copy(dst_hbm.at[wid], didx)
    plsc.subcore_barrier()

    def drain():
      pltpu.make_async_copy(ones, acc.at[didx.at[0]], ssem).wait()

    @pl.loop(0, NB)
    def _edge(j):
      pltpu.async_copy(ones, acc.at[didx.at[j]], ssem, add=True)

      @pl.when(j >= 16)
      def _lag():
        drain()

    @pl.loop(0, 16)
    def _tail(j):
      drain()

    plsc.subcore_barrier()
    for k in range(RPT // 128):
      sl = pl.ds(s * RPT + k * 128, 128)
      pltpu.sync_copy(acc.at[sl], out_hbm.at[c, sl])

  return pl.kernel(
      body,
      out_type=jax.ShapeDtypeStruct((NC, NPAD, 16), jnp.float32),
      mesh=_vector_mesh,
      compiler_params=pltpu.CompilerParams(use_tc_tiling_on_sc=False),
      scratch_types=[
          pltpu.VMEM((NB, B), jnp.int32),
          pltpu.VMEM((B, 16), jnp.float32),
          pltpu.VMEM((128, 16), jnp.float32),
          pltpu.VMEM_SHARED((NPAD, 16), jnp.float32),
          pltpu.SemaphoreType.DMA,
      ],
  )


_agg48 = _make_agg(F1)
_agg32 = _make_agg(F2)
_deg = _make_deg()


BR = 2000


def _row_spec(f):
  return pl.BlockSpec((BR, f), lambda i: (i, 0))


def _agg_spec(f):
  return pl.BlockSpec((2, BR, f), lambda i: (0, i, 0))


def _full_spec(shape):
  return pl.BlockSpec(shape, lambda i: tuple(0 for _ in shape))


def _k1a_body(x_ref, w_ref, xw_ref):
  xw_ref[...] = jnp.dot(x_ref[...], w_ref[...],
                        preferred_element_type=jnp.float32)


def _k1a(x, w1p):
  return pl.pallas_call(
      _k1a_body,
      grid=(N // BR,),
      in_specs=[_row_spec(128), _full_spec((128, F1))],
      out_specs=_row_spec(F1),
      out_shape=jax.ShapeDtypeStruct((N, F1), jnp.float32),
  )(x, w1p)


def _k1b_body(xw_ref, dp_ref, y_ref):
  deg = dp_ref[0, :, 0:1] + dp_ref[1, :, 0:1] + 1.0
  dinv = lax.rsqrt(deg)
  col = lax.broadcasted_iota(jnp.int32, (BR, F1), 1)
  y_ref[...] = jnp.where(col == F1 - 1, dinv, xw_ref[...] * dinv)


def _k1b(xw, degp):
  return pl.pallas_call(
      _k1b_body,
      grid=(N // BR,),
      in_specs=[_row_spec(F1), _agg_spec(16)],
      out_specs=_row_spec(F1),
      out_shape=jax.ShapeDtypeStruct((N, F1), jnp.float32),
  )(xw, degp)


def _k3_body(a_ref, y_ref, b_ref, w_ref, o_ref):
  y = y_ref[...]
  dinv = y[:, F1 - 1:F1]
  asum = a_ref[0] + a_ref[1]
  h = jnp.maximum(dinv * (asum + y) + b_ref[...], 0.0)
  o_ref[...] = jnp.dot(h, w_ref[...], preferred_element_type=jnp.float32) * dinv


def _k3(a, y, b, w):
  fout = w.shape[1]
  return pl.pallas_call(
      _k3_body,
      grid=(N // BR,),
      in_specs=[_agg_spec(F1), _row_spec(F1), _full_spec((1, F1)),
                _full_spec((F1, fout))],
      out_specs=_row_spec(fout),
      out_shape=jax.ShapeDtypeStruct((N, fout), jnp.float32),
  )(a, y, b, w)


def _k5_body(a_ref, y_ref, d_ref, b_ref, o_ref):
  dinv = d_ref[:, F1 - 1:F1]
  asum = a_ref[0] + a_ref[1]
  h = jnp.maximum(dinv * (asum + y_ref[...]) + b_ref[...], 0.0)
  o_ref[...] = h * dinv


def _k5(a, y, y1, b):
  f = y.shape[1]
  return pl.pallas_call(
      _k5_body,
      grid=(N // BR,),
      in_specs=[_agg_spec(f), _row_spec(f), _row_spec(F1),
                _full_spec((1, f))],
      out_specs=_row_spec(f),
      out_shape=jax.ShapeDtypeStruct((N, f), jnp.float32),
  )(a, y, y1, b)


def _k7_body(a_ref, y_ref, d_ref, wmu_ref, bmu_ref, wlv_ref,
             blv_ref, mu_ref, lv_ref):
  dinv = d_ref[:, F1 - 1:F1]
  asum = a_ref[0] + a_ref[1]
  g = dinv * (asum + y_ref[...])
  mu_ref[...] = jnp.dot(g, wmu_ref[...],
                        preferred_element_type=jnp.float32) + bmu_ref[...]
  lv_ref[...] = jnp.dot(g, wlv_ref[...],
                        preferred_element_type=jnp.float32) + blv_ref[...]


def _k7(a, y, y1, wmu, bmu, wlv, blv):
  f = y.shape[1]
  fout = wmu.shape[1]
  return pl.pallas_call(
      _k7_body,
      grid=(N // BR,),
      in_specs=[_agg_spec(f), _row_spec(f), _row_spec(F1),
                _full_spec((f, fout)), _full_spec((1, fout)),
                _full_spec((f, fout)), _full_spec((1, fout))],
      out_specs=[_row_spec(fout), _row_spec(fout)],
      out_shape=[
          jax.ShapeDtypeStruct((N, fout), jnp.float32),
          jax.ShapeDtypeStruct((N, fout), jnp.float32),
      ],
  )(a, y, y1, wmu, bmu, wlv, blv)


def kernel(x, edge_index, W1, b1, W2, b2, Wmu, bmu, Wlv, blv):
  eidx = edge_index.astype(jnp.int32).reshape(2, NW, NB, B)
  srcp = eidx[0]
  dstp = eidx[1]

  degp = _deg(dstp)

  f32 = jnp.float32
  w1p = jnp.pad(W1.astype(f32), ((0, 0), (0, F1 - W1.shape[1])))
  b1p = jnp.pad(b1.astype(f32), (0, F1 - b1.shape[0])).reshape(1, F1)
  w2p = jnp.pad(W2.astype(f32), ((0, F1 - W2.shape[0]), (0, 0)))

  xw1 = _k1a(x, w1p)
  y1 = _k1b(xw1, degp)
  a1 = _agg48(y1, srcp, dstp)
  y2 = _k3(a1, y1, b1p, w2p)
  a2 = _agg32(y2, srcp, dstp)
  y3 = _k5(a2, y2, y1, b2.reshape(1, F2))
  a3 = _agg32(y3, srcp, dstp)
  mu, lv = _k7(a3, y3, y1,
               Wmu, bmu.reshape(1, -1), Wlv, blv.reshape(1, -1))
  return (mu, lv)


# speedup vs baseline: 1.0148x; 1.0148x over previous
import functools

import jax
import jax.numpy as jnp
from jax import lax
from jax.experimental import pallas as pl
from jax.experimental.pallas import tpu as pltpu
from jax.experimental.pallas import tpu_sc as plsc

N = 10000
E = 320000
NC = 2
NS = 16
NW = NC * NS
B = 80
NB = 125
D = 5
NPAD = 10240
RPT = NPAD // NS
F1 = 48
F2 = 32

_vector_mesh = plsc.VectorSubcoreMesh(core_axis_name="c", subcore_axis_name="s")


def _make_agg(F):
  def body(y_hbm, src_hbm, dst_hbm, out_hbm, sidx, didx, b0, b1, b2, b3, b4,
           zbuf, ysp, acc, g0, g1, g2, g3, g4, s0, s1, s2, s3, s4):
    c = lax.axis_index("c")
    s = lax.axis_index("s")
    wid = c * NS + s
    bufs = [b0, b1, b2, b3, b4]
    gsems = [g0, g1, g2, g3, g4]
    ssems = [s0, s1, s2, s3, s4]
    zero16 = jnp.zeros((16,), jnp.float32)

    @pl.loop(0, 128)
    def _zero(i):
      for j in range(F // 16):
        zbuf[i, pl.ds(j * 16, 16)] = zero16

    for k in range(RPT // 128):
      pltpu.sync_copy(zbuf, acc.at[pl.ds(s * RPT + k * 128, 128)])

    ycp = pltpu.async_copy(y_hbm.at[pl.ds(s * (N // NS), N // NS)],
                           ysp.at[pl.ds(s * (N // NS), N // NS)], g0)
    pltpu.sync_copy(src_hbm.at[wid], sidx)
    pltpu.sync_copy(dst_hbm.at[wid], didx)
    ycp.wait()
    plsc.subcore_barrier()

    def gather(j, t):
      return pltpu.make_async_copy(ysp.at[sidx.at[j]], bufs[t], gsems[t])

    def scatter(j, t):
      return pltpu.async_copy(bufs[t], acc.at[didx.at[j]], ssems[t], add=True)

    def drain_scatter(t):
      pltpu.make_async_copy(bufs[t], acc.at[didx.at[0]], ssems[t]).wait()

    pltpu.async_copy(ysp.at[sidx.at[0]], bufs[0], gsems[0])

    @pl.loop(0, NB // D)
    def _grp(k):
      j0 = k * D
      for t in range(D):
        tn = (t + 1) % D
        j = j0 + t
        jn = j + 1

        @pl.when(jn < NB)
        def _pref():
          @pl.when(jn >= D)
          def _drain():
            drain_scatter(tn)
          pltpu.async_copy(ysp.at[sidx.at[jn]], bufs[tn], gsems[tn])

        gather(j, t).wait()
        scatter(j, t)

    for t in range(D):
      drain_scatter(t)

    plsc.subcore_barrier()
    for k in range(RPT // 128):
      sl = pl.ds(s * RPT + k * 128, 128)
      pltpu.sync_copy(acc.at[sl], out_hbm.at[c, sl])

  return pl.kernel(
      body,
      out_type=jax.ShapeDtypeStruct((NC, NPAD, F), jnp.float32),
      mesh=_vector_mesh,
      compiler_params=pltpu.CompilerParams(use_tc_tiling_on_sc=False),
      scratch_types=(
          [pltpu.VMEM((NB, B), jnp.int32),
           pltpu.VMEM((NB, B), jnp.int32)] +
          [pltpu.VMEM((B, F), jnp.float32)] * D +
          [pltpu.VMEM((128, F), jnp.float32),
           pltpu.VMEM_SHARED((N, F), jnp.float32),
           pltpu.VMEM_SHARED((NPAD, F), jnp.float32)] +
          [pltpu.SemaphoreType.DMA] * (2 * D)
      ),
  )


def _make_deg():
  def body(dst_hbm, out_hbm, didx, ones, zbuf, acc, ssem):
    c = lax.axis_index("c")
    s = lax.axis_index("s")
    wid = c * NS + s
    zero16 = jnp.zeros((16,), jnp.float32)
    one16 = jnp.ones((16,), jnp.float32)

    @pl.loop(0, 128)
    def _fill(i):
      zbuf[i] = zero16

    @pl.loop(0, B)
    def _fill1(i):
      ones[i] = one16

    for k in range(RPT // 128):
      pltpu.sync_copy(zbuf, acc.at[pl.ds(s * RPT + k * 128, 128)])

    pltpu.sync_copy(dst_hbm.at[wid], didx)
    plsc.subcore_barrier()

    def drain():
      pltpu.make_async_copy(ones, acc.at[didx.at[0]], ssem).wait()

    @pl.loop(0, NB)
    def _edge(j):
      pltpu.async_copy(ones, acc.at[didx.at[j]], ssem, add=True)

      @pl.when(j >= 16)
      def _lag():
        drain()

    @pl.loop(0, 16)
    def _tail(j):
      drain()

    plsc.subcore_barrier()
    for k in range(RPT // 128):
      sl = pl.ds(s * RPT + k * 128, 128)
      pltpu.sync_copy(acc.at[sl], out_hbm.at[c, sl])

  return pl.kernel(
      body,
      out_type=jax.ShapeDtypeStruct((NC, NPAD, 16), jnp.float32),
      mesh=_vector_mesh,
      compiler_params=pltpu.CompilerParams(use_tc_tiling_on_sc=False),
      scratch_types=[
          pltpu.VMEM((NB, B), jnp.int32),
          pltpu.VMEM((B, 16), jnp.float32),
          pltpu.VMEM((128, 16), jnp.float32),
          pltpu.VMEM_SHARED((NPAD, 16), jnp.float32),
          pltpu.SemaphoreType.DMA,
      ],
  )


_agg48 = _make_agg(F1)
_agg32 = _make_agg(F2)
_deg = _make_deg()


BR = 2000


def _row_spec(f):
  return pl.BlockSpec((BR, f), lambda i: (i, 0))


def _agg_spec(f):
  return pl.BlockSpec((2, BR, f), lambda i: (0, i, 0))


def _full_spec(shape):
  return pl.BlockSpec(shape, lambda i: tuple(0 for _ in shape))


def _k1a_body(x_ref, w_ref, xw_ref):
  xw_ref[...] = jnp.dot(x_ref[...], w_ref[...],
                        preferred_element_type=jnp.float32)


def _k1a(x, w1p):
  return pl.pallas_call(
      _k1a_body,
      grid=(N // BR,),
      in_specs=[_row_spec(128), _full_spec((128, F1))],
      out_specs=_row_spec(F1),
      out_shape=jax.ShapeDtypeStruct((N, F1), jnp.float32),
  )(x, w1p)


def _k1b_body(xw_ref, dp_ref, y_ref):
  deg = dp_ref[0, :, 0:1] + dp_ref[1, :, 0:1] + 1.0
  dinv = lax.rsqrt(deg)
  col = lax.broadcasted_iota(jnp.int32, (BR, F1), 1)
  y_ref[...] = jnp.where(col == F1 - 1, dinv, xw_ref[...] * dinv)


def _k1b(xw, degp):
  return pl.pallas_call(
      _k1b_body,
      grid=(N // BR,),
      in_specs=[_row_spec(F1), _agg_spec(16)],
      out_specs=_row_spec(F1),
      out_shape=jax.ShapeDtypeStruct((N, F1), jnp.float32),
  )(xw, degp)


def _k3_body(a_ref, y_ref, b_ref, w_ref, o_ref):
  y = y_ref[...]
  dinv = y[:, F1 - 1:F1]
  asum = a_ref[0] + a_ref[1]
  h = jnp.maximum(dinv * (asum + y) + b_ref[...], 0.0)
  o_ref[...] = jnp.dot(h, w_ref[...], preferred_element_type=jnp.float32) * dinv


def _k3(a, y, b, w):
  fout = w.shape[1]
  return pl.pallas_call(
      _k3_body,
      grid=(N // BR,),
      in_specs=[_agg_spec(F1), _row_spec(F1), _full_spec((1, F1)),
                _full_spec((F1, fout))],
      out_specs=_row_spec(fout),
      out_shape=jax.ShapeDtypeStruct((N, fout), jnp.float32),
  )(a, y, b, w)


def _k5_body(a_ref, y_ref, d_ref, b_ref, o_ref):
  dinv = d_ref[:, F1 - 1:F1]
  asum = a_ref[0] + a_ref[1]
  h = jnp.maximum(dinv * (asum + y_ref[...]) + b_ref[...], 0.0)
  o_ref[...] = h * dinv


def _k5(a, y, y1, b):
  f = y.shape[1]
  return pl.pallas_call(
      _k5_body,
      grid=(N // BR,),
      in_specs=[_agg_spec(f), _row_spec(f), _row_spec(F1),
                _full_spec((1, f))],
      out_specs=_row_spec(f),
      out_shape=jax.ShapeDtypeStruct((N, f), jnp.float32),
  )(a, y, y1, b)


def _k7_body(a_ref, y_ref, d_ref, wmu_ref, bmu_ref, wlv_ref,
             blv_ref, mu_ref, lv_ref):
  dinv = d_ref[:, F1 - 1:F1]
  asum = a_ref[0] + a_ref[1]
  g = dinv * (asum + y_ref[...])
  mu_ref[...] = jnp.dot(g, wmu_ref[...],
                        preferred_element_type=jnp.float32) + bmu_ref[...]
  lv_ref[...] = jnp.dot(g, wlv_ref[...],
                        preferred_element_type=jnp.float32) + blv_ref[...]


def _k7(a, y, y1, wmu, bmu, wlv, blv):
  f = y.shape[1]
  fout = wmu.shape[1]
  return pl.pallas_call(
      _k7_body,
      grid=(N // BR,),
      in_specs=[_agg_spec(f), _row_spec(f), _row_spec(F1),
                _full_spec((f, fout)), _full_spec((1, fout)),
                _full_spec((f, fout)), _full_spec((1, fout))],
      out_specs=[_row_spec(fout), _row_spec(fout)],
      out_shape=[
          jax.ShapeDtypeStruct((N, fout), jnp.float32),
          jax.ShapeDtypeStruct((N, fout), jnp.float32),
      ],
  )(a, y, y1, wmu, bmu, wlv, blv)


def kernel(x, edge_index, W1, b1, W2, b2, Wmu, bmu, Wlv, blv):
  srcp = edge_index[0].astype(jnp.int32).reshape(NW, NB, B)
  dstp = edge_index[1].astype(jnp.int32).reshape(NW, NB, B)

  degp = _deg(dstp)

  f32 = jnp.float32
  w1p = jnp.pad(W1.astype(f32), ((0, 0), (0, F1 - W1.shape[1])))
  b1p = jnp.pad(b1.astype(f32), (0, F1 - b1.shape[0])).reshape(1, F1)
  w2p = jnp.pad(W2.astype(f32), ((0, F1 - W2.shape[0]), (0, 0)))

  xw1 = _k1a(x, w1p)
  y1 = _k1b(xw1, degp)
  a1 = _agg48(y1, srcp, dstp)
  y2 = _k3(a1, y1, b1p, w2p)
  a2 = _agg32(y2, srcp, dstp)
  y3 = _k5(a2, y2, y1, b2.reshape(1, F2))
  a3 = _agg32(y3, srcp, dstp)
  mu, lv = _k7(a3, y3, y1,
               Wmu, bmu.reshape(1, -1), Wlv, blv.reshape(1, -1))
  return (mu, lv)
